# baseline (speedup 1.0000x reference)
import functools

import jax
import jax.numpy as jnp
import numpy as np
from jax import lax
from jax.experimental import pallas as pl
from jax.experimental.pallas import tpu as pltpu
from jax.experimental.pallas import tpu_sc as plsc

_M16 = 0xFFFF
_M32 = 0xFFFFFFFF

_K_MUL = 0xC6A4A7935BD1E995
_SM1 = 0x9E3779B97F4A7C15
_SM2 = 0xBF58476D1CE4E5B9
_SM3 = 0x94D049BB133111EB
_DEFAULT_HASH_KEY = 0xDECAFCAFFE


def _py_fp64(x):
    x = (x + _SM1) & 0xFFFFFFFFFFFFFFFF
    x = ((x ^ (x >> 30)) * _SM2) & 0xFFFFFFFFFFFFFFFF
    x = ((x ^ (x >> 27)) * _SM3) & 0xFFFFFFFFFFFFFFFF
    return x ^ (x >> 31)


_C0 = _py_fp64(_DEFAULT_HASH_KEY) ^ _K_MUL

_MAGIC = (1 << 45) // 15625 + 1

_N = 16384
_L = 16
_NC = 2
_NS = 16
_NW = _NC * _NS
_CHUNK = _N // _NW
_NG = _CHUNK // _L
_UNROLL = 1


def _u32(c):
    return jnp.uint32(c & _M32)


def _mul64_const(ah, al, b):
    b_hi = (b >> 32) & _M32
    b_lo = b & _M32
    a0 = al & _u32(_M16)
    a1 = al >> 16
    ll = a0 * _u32(b_lo & _M16)
    lh = a0 * _u32(b_lo >> 16)
    hl = a1 * _u32(b_lo & _M16)
    hh = a1 * _u32(b_lo >> 16)
    mid = lh + hl
    c_mid = jnp.where(mid < lh, _u32(1 << 16), _u32(0))
    lo = ll + (mid << 16)
    c_lo = jnp.where(lo < ll, _u32(1), _u32(0))
    hi = (hh + (mid >> 16) + c_mid + c_lo
          + al * _u32(b_hi) + ah * _u32(b_lo))
    return hi, lo


def _mul64_const_ahc(ah_const, al, b):
    b_hi = (b >> 32) & _M32
    b_lo = b & _M32
    a0 = al & _u32(_M16)
    a1 = al >> 16
    ll = a0 * _u32(b_lo & _M16)
    lh = a0 * _u32(b_lo >> 16)
    hl = a1 * _u32(b_lo & _M16)
    hh = a1 * _u32(b_lo >> 16)
    mid = lh + hl
    c_mid = jnp.where(mid < lh, _u32(1 << 16), _u32(0))
    lo = ll + (mid << 16)
    c_lo = jnp.where(lo < ll, _u32(1), _u32(0))
    hi = (hh + (mid >> 16) + c_mid + c_lo
          + al * _u32(b_hi) + _u32(ah_const * b_lo))
    return hi, lo


def _xorshr(hi, lo, s):
    return hi ^ (hi >> s), lo ^ ((lo >> s) | (hi << (32 - s)))


_SM1_LO = _SM1 & _M32
_SM1_HI = _SM1 >> 32
_FP_H1 = _SM1_HI ^ (_SM1_HI >> 30)
_FP_LOX = (_SM1_HI << 2) & _M32


def _fp64_small(vl):
    lo = vl + _u32(_SM1_LO)
    lo = lo ^ ((lo >> 30) | _u32(_FP_LOX))
    hi, lo = _mul64_const_ahc(_FP_H1, lo, _SM2)
    hi, lo = _xorshr(hi, lo, 27)
    hi, lo = _mul64_const(hi, lo, _SM3)
    return _xorshr(hi, lo, 31)


def _cat64(ch, cl, fh, fl):
    mh, ml = _mul64_const(fh, fl, _K_MUL)
    ml = ml ^ (mh >> 15)
    mh, ml = _mul64_const(mh, ml, _K_MUL)
    rh = ch ^ mh
    rl = cl ^ ml
    rh, rl = _mul64_const(rh, rl, _K_MUL)
    rl = rl ^ (rh >> 15)
    return _mul64_const(rh, rl, _K_MUL)


_RCP = float(np.nextafter(np.float32(1.0) / np.float32(15625.0),
                          np.float32(0.0)))
_R16 = (1 << 16) % 15625


def _mod1e6(hi, lo):
    def step(r, d):
        w = r * _u32(_R16) + d
        wf = lax.bitcast_convert_type(w, jnp.int32).astype(jnp.float32)
        q = (wf * jnp.float32(_RCP)).astype(jnp.int32)
        rr = w - lax.bitcast_convert_type(q, jnp.uint32) * _u32(15625)
        return jnp.where(rr >= _u32(15625), rr - _u32(15625), rr)

    r = step(jnp.zeros_like(hi), hi >> 16)
    r = step(r, hi & _u32(_M16))
    r = step(r, lo >> 16)
    r = step(r, lo & _u32(_M16))
    a = lo & _u32(63)
    t = ((a - r) * _u32(57)) & _u32(63)
    return r + t * _u32(15625)


def _hash16(l0, l1, l2):
    fh, fl = _fp64_small(l0)
    ch, cl = _cat64(_u32(_C0 >> 32), _u32(_C0), fh, fl)
    fh, fl = _fp64_small(l1)
    ch, cl = _cat64(ch ^ _u32(_K_MUL >> 32), cl ^ _u32(_K_MUL), fh, fl)
    fh, fl = _fp64_small(l2)
    ch, cl = _cat64(ch ^ _u32(_K_MUL >> 32), cl ^ _u32(_K_MUL), fh, fl)
    return _mod1e6(ch, cl)


def _sc_body(in_hbm, out_hbm, v0, v1, v2, vout, sem):
    wid = lax.axis_index("s") * _NC + lax.axis_index("c")
    base = wid * _CHUNK
    sl = pl.ds(base, _CHUNK)
    copies = [
        pltpu.async_copy(in_hbm.at[pl.ds(base, _CHUNK)], v0, sem),
        pltpu.async_copy(in_hbm.at[pl.ds(base + jnp.int32(_N), _CHUNK)], v1, sem),
        pltpu.async_copy(in_hbm.at[pl.ds(base + jnp.int32(2 * _N), _CHUNK)], v2, sem),
    ]
    for c in copies:
        c.wait()

    def body(g, carry):
        for u in range(_UNROLL):
            off = g * jnp.int32(_UNROLL * _L) + jnp.int32(u * _L)
            idx = pl.ds(pl.multiple_of(off, _L), _L)
            vout[idx] = _hash16(v0[idx], v1[idx], v2[idx])
        return carry

    lax.fori_loop(jnp.int32(0), jnp.int32(_NG // _UNROLL), body, 0)
    pltpu.sync_copy(vout, out_hbm.at[sl])


@jax.jit
def _crossing(packed):
    run = functools.partial(
        pl.kernel,
        mesh=plsc.VectorSubcoreMesh(core_axis_name="c", subcore_axis_name="s"),
        out_type=jax.ShapeDtypeStruct((_N,), jnp.uint32),
        scratch_types=[pltpu.VMEM((_CHUNK,), jnp.uint32)] * 4
        + [pltpu.SemaphoreType.DMA],
    )(_sc_body)
    return run(packed)


def kernel(inp_0, inp_1, inp_2):
    parts = []
    for inp in (inp_0, inp_1, inp_2):
        lo = lax.convert_element_type(inp.reshape(_N), jnp.int32)
        parts.append(lax.bitcast_convert_type(lo, jnp.uint32))
    out = _crossing(jnp.concatenate(parts))
    return out.astype(jnp.int64).reshape(_N, 1)


# speedup vs baseline: 1.0116x; 1.0116x over previous
import functools

import jax
import jax.numpy as jnp
import numpy as np
from jax import lax
from jax.experimental import pallas as pl
from jax.experimental.pallas import tpu as pltpu
from jax.experimental.pallas import tpu_sc as plsc

_M16 = 0xFFFF
_M32 = 0xFFFFFFFF

_K_MUL = 0xC6A4A7935BD1E995
_SM1 = 0x9E3779B97F4A7C15
_SM2 = 0xBF58476D1CE4E5B9
_SM3 = 0x94D049BB133111EB
_DEFAULT_HASH_KEY = 0xDECAFCAFFE


def _py_fp64(x):
    x = (x + _SM1) & 0xFFFFFFFFFFFFFFFF
    x = ((x ^ (x >> 30)) * _SM2) & 0xFFFFFFFFFFFFFFFF
    x = ((x ^ (x >> 27)) * _SM3) & 0xFFFFFFFFFFFFFFFF
    return x ^ (x >> 31)


_C0 = _py_fp64(_DEFAULT_HASH_KEY) ^ _K_MUL

_MAGIC = (1 << 45) // 15625 + 1

_N = 16384
_L = 16
_NC = 2
_NS = 16
_NW = _NC * _NS
_CHUNK = _N // _NW
_NG = _CHUNK // _L
_UNROLL = 1


def _u32(c):
    return jnp.uint32(c & _M32)


def _mul64_const(ah, al, b):
    b_hi = (b >> 32) & _M32
    b_lo = b & _M32
    a0 = al & _u32(_M16)
    a1 = al >> 16
    ll = a0 * _u32(b_lo & _M16)
    lh = a0 * _u32(b_lo >> 16)
    hl = a1 * _u32(b_lo & _M16)
    hh = a1 * _u32(b_lo >> 16)
    mid = lh + hl
    c_mid = jnp.where(mid < lh, _u32(1 << 16), _u32(0))
    lo = ll + (mid << 16)
    c_lo = jnp.where(lo < ll, _u32(1), _u32(0))
    hi = (hh + (mid >> 16) + c_mid + c_lo
          + al * _u32(b_hi) + ah * _u32(b_lo))
    return hi, lo


def _mul64_const_ahc(ah_const, al, b):
    b_hi = (b >> 32) & _M32
    b_lo = b & _M32
    a0 = al & _u32(_M16)
    a1 = al >> 16
    ll = a0 * _u32(b_lo & _M16)
    lh = a0 * _u32(b_lo >> 16)
    hl = a1 * _u32(b_lo & _M16)
    hh = a1 * _u32(b_lo >> 16)
    mid = lh + hl
    c_mid = jnp.where(mid < lh, _u32(1 << 16), _u32(0))
    lo = ll + (mid << 16)
    c_lo = jnp.where(lo < ll, _u32(1), _u32(0))
    hi = (hh + (mid >> 16) + c_mid + c_lo
          + al * _u32(b_hi) + _u32(ah_const * b_lo))
    return hi, lo


def _xorshr(hi, lo, s):
    return hi ^ (hi >> s), lo ^ ((lo >> s) | (hi << (32 - s)))


_SM1_LO = _SM1 & _M32
_SM1_HI = _SM1 >> 32
_FP_H1 = _SM1_HI ^ (_SM1_HI >> 30)
_FP_LOX = (_SM1_HI << 2) & _M32


def _fp64_small(vl):
    lo = vl + _u32(_SM1_LO)
    lo = lo ^ ((lo >> 30) | _u32(_FP_LOX))
    hi, lo = _mul64_const_ahc(_FP_H1, lo, _SM2)
    hi, lo = _xorshr(hi, lo, 27)
    hi, lo = _mul64_const(hi, lo, _SM3)
    return _xorshr(hi, lo, 31)


def _cat64(ch, cl, fh, fl):
    mh, ml = _mul64_const(fh, fl, _K_MUL)
    ml = ml ^ (mh >> 15)
    mh, ml = _mul64_const(mh, ml, _K_MUL)
    rh = ch ^ mh
    rl = cl ^ ml
    rh, rl = _mul64_const(rh, rl, _K_MUL)
    rl = rl ^ (rh >> 15)
    return _mul64_const(rh, rl, _K_MUL)


def _mod1e6(hi, lo):
    def step(r, d):
        v = (r << 16) | d
        v0 = v & _u32(_M16)
        v1 = v >> 16
        ll = v0 * _u32(_MAGIC & _M16)
        lh = v0 * _u32(_MAGIC >> 16)
        hl = v1 * _u32(_MAGIC & _M16)
        hh = v1 * _u32(_MAGIC >> 16)
        mid = lh + hl
        c_mid = jnp.where(mid < lh, _u32(1 << 16), _u32(0))
        plo = ll + (mid << 16)
        c_lo = jnp.where(plo < ll, _u32(1), _u32(0))
        phi = hh + (mid >> 16) + c_mid + c_lo
        q = phi >> 13
        return v - q * _u32(15625)

    r = step(jnp.zeros_like(hi), hi >> 16)
    r = step(r, hi & _u32(_M16))
    r = step(r, lo >> 16)
    r = step(r, lo & _u32(_M16))
    a = lo & _u32(63)
    t = ((a - r) * _u32(57)) & _u32(63)
    return r + t * _u32(15625)


def _hash16(l0, l1, l2):
    fh, fl = _fp64_small(l0)
    ch, cl = _cat64(_u32(_C0 >> 32), _u32(_C0), fh, fl)
    fh, fl = _fp64_small(l1)
    ch, cl = _cat64(ch ^ _u32(_K_MUL >> 32), cl ^ _u32(_K_MUL), fh, fl)
    fh, fl = _fp64_small(l2)
    ch, cl = _cat64(ch ^ _u32(_K_MUL >> 32), cl ^ _u32(_K_MUL), fh, fl)
    return _mod1e6(ch, cl)


def _sc_body(in_hbm, out_hbm, v0, v1, v2, vout, sem):
    wid = lax.axis_index("s") * _NC + lax.axis_index("c")
    base = wid * _CHUNK
    sl = pl.ds(base, _CHUNK)
    copies = [
        pltpu.async_copy(in_hbm.at[pl.ds(base, _CHUNK)], v0, sem),
        pltpu.async_copy(in_hbm.at[pl.ds(base + jnp.int32(_N), _CHUNK)], v1, sem),
        pltpu.async_copy(in_hbm.at[pl.ds(base + jnp.int32(2 * _N), _CHUNK)], v2, sem),
    ]
    for c in copies:
        c.wait()

    def body(g, carry):
        for u in range(_UNROLL):
            off = g * jnp.int32(_UNROLL * _L) + jnp.int32(u * _L)
            idx = pl.ds(pl.multiple_of(off, _L), _L)
            vout[idx] = _hash16(v0[idx], v1[idx], v2[idx])
        return carry

    lax.fori_loop(jnp.int32(0), jnp.int32(_NG // _UNROLL), body, 0)
    pltpu.sync_copy(vout, out_hbm.at[sl])


@jax.jit
def _crossing(packed):
    run = functools.partial(
        pl.kernel,
        mesh=plsc.VectorSubcoreMesh(core_axis_name="c", subcore_axis_name="s"),
        out_type=jax.ShapeDtypeStruct((_N,), jnp.uint32),
        scratch_types=[pltpu.VMEM((_CHUNK,), jnp.uint32)] * 4
        + [pltpu.SemaphoreType.DMA],
    )(_sc_body)
    return run(packed)


def kernel(inp_0, inp_1, inp_2):
    parts = []
    for inp in (inp_0, inp_1, inp_2):
        lo = lax.convert_element_type(inp.reshape(_N), jnp.int32)
        parts.append(lax.bitcast_convert_type(lo, jnp.uint32))
    out = _crossing(jnp.concatenate(parts))
    return out.astype(jnp.int64).reshape(_N, 1)


# speedup vs baseline: 1.0791x; 1.0668x over previous
import functools

import jax
import jax.numpy as jnp
import numpy as np
from jax import lax
from jax.experimental import pallas as pl
from jax.experimental.pallas import tpu as pltpu
from jax.experimental.pallas import tpu_sc as plsc

_M16 = 0xFFFF
_M32 = 0xFFFFFFFF

_K_MUL = 0xC6A4A7935BD1E995
_SM1 = 0x9E3779B97F4A7C15
_SM2 = 0xBF58476D1CE4E5B9
_SM3 = 0x94D049BB133111EB
_DEFAULT_HASH_KEY = 0xDECAFCAFFE


def _py_fp64(x):
    x = (x + _SM1) & 0xFFFFFFFFFFFFFFFF
    x = ((x ^ (x >> 30)) * _SM2) & 0xFFFFFFFFFFFFFFFF
    x = ((x ^ (x >> 27)) * _SM3) & 0xFFFFFFFFFFFFFFFF
    return x ^ (x >> 31)


_C0 = _py_fp64(_DEFAULT_HASH_KEY) ^ _K_MUL

_MAGIC = (1 << 45) // 15625 + 1

_N = 16384
_L = 16
_NC = 2
_NS = 16
_NW = _NC * _NS
_S = 4096
_T = _N - _S
_CHUNK = _S // _NW
_NG = _CHUNK // _L
_UNROLL = 1


def _u32(c):
    return jnp.uint32(c & _M32)


def _mul64_const(ah, al, b):
    b_hi = (b >> 32) & _M32
    b_lo = b & _M32
    a0 = al & _u32(_M16)
    a1 = al >> 16
    ll = a0 * _u32(b_lo & _M16)
    lh = a0 * _u32(b_lo >> 16)
    hl = a1 * _u32(b_lo & _M16)
    hh = a1 * _u32(b_lo >> 16)
    mid = lh + hl
    c_mid = jnp.where(mid < lh, _u32(1 << 16), _u32(0))
    lo = ll + (mid << 16)
    c_lo = jnp.where(lo < ll, _u32(1), _u32(0))
    hi = (hh + (mid >> 16) + c_mid + c_lo
          + al * _u32(b_hi) + ah * _u32(b_lo))
    return hi, lo


def _mul64_const_ahc(ah_const, al, b):
    b_hi = (b >> 32) & _M32
    b_lo = b & _M32
    a0 = al & _u32(_M16)
    a1 = al >> 16
    ll = a0 * _u32(b_lo & _M16)
    lh = a0 * _u32(b_lo >> 16)
    hl = a1 * _u32(b_lo & _M16)
    hh = a1 * _u32(b_lo >> 16)
    mid = lh + hl
    c_mid = jnp.where(mid < lh, _u32(1 << 16), _u32(0))
    lo = ll + (mid << 16)
    c_lo = jnp.where(lo < ll, _u32(1), _u32(0))
    hi = (hh + (mid >> 16) + c_mid + c_lo
          + al * _u32(b_hi) + _u32(ah_const * b_lo))
    return hi, lo


def _xorshr(hi, lo, s):
    return hi ^ (hi >> s), lo ^ ((lo >> s) | (hi << (32 - s)))


_SM1_LO = _SM1 & _M32
_SM1_HI = _SM1 >> 32
_FP_H1 = _SM1_HI ^ (_SM1_HI >> 30)
_FP_LOX = (_SM1_HI << 2) & _M32


def _fp64_small(vl):
    lo = vl + _u32(_SM1_LO)
    lo = lo ^ ((lo >> 30) | _u32(_FP_LOX))
    hi, lo = _mul64_const_ahc(_FP_H1, lo, _SM2)
    hi, lo = _xorshr(hi, lo, 27)
    hi, lo = _mul64_const(hi, lo, _SM3)
    return _xorshr(hi, lo, 31)


def _cat64(ch, cl, fh, fl):
    mh, ml = _mul64_const(fh, fl, _K_MUL)
    ml = ml ^ (mh >> 15)
    mh, ml = _mul64_const(mh, ml, _K_MUL)
    rh = ch ^ mh
    rl = cl ^ ml
    rh, rl = _mul64_const(rh, rl, _K_MUL)
    rl = rl ^ (rh >> 15)
    return _mul64_const(rh, rl, _K_MUL)


def _mod1e6(hi, lo):
    def step(r, d):
        v = (r << 16) | d
        v0 = v & _u32(_M16)
        v1 = v >> 16
        ll = v0 * _u32(_MAGIC & _M16)
        lh = v0 * _u32(_MAGIC >> 16)
        hl = v1 * _u32(_MAGIC & _M16)
        hh = v1 * _u32(_MAGIC >> 16)
        mid = lh + hl
        c_mid = jnp.where(mid < lh, _u32(1 << 16), _u32(0))
        plo = ll + (mid << 16)
        c_lo = jnp.where(plo < ll, _u32(1), _u32(0))
        phi = hh + (mid >> 16) + c_mid + c_lo
        q = phi >> 13
        return v - q * _u32(15625)

    r = step(jnp.zeros_like(hi), hi >> 16)
    r = step(r, hi & _u32(_M16))
    r = step(r, lo >> 16)
    r = step(r, lo & _u32(_M16))
    a = lo & _u32(63)
    t = ((a - r) * _u32(57)) & _u32(63)
    return r + t * _u32(15625)


def _hash16(l0, l1, l2):
    fh, fl = _fp64_small(l0)
    ch, cl = _cat64(_u32(_C0 >> 32), _u32(_C0), fh, fl)
    fh, fl = _fp64_small(l1)
    ch, cl = _cat64(ch ^ _u32(_K_MUL >> 32), cl ^ _u32(_K_MUL), fh, fl)
    fh, fl = _fp64_small(l2)
    ch, cl = _cat64(ch ^ _u32(_K_MUL >> 32), cl ^ _u32(_K_MUL), fh, fl)
    return _mod1e6(ch, cl)


def _sc_body(in_hbm, out_hbm, v0, v1, v2, vout, sem):
    wid = lax.axis_index("s") * _NC + lax.axis_index("c")
    base = wid * _CHUNK
    sl = pl.ds(base, _CHUNK)
    copies = [
        pltpu.async_copy(in_hbm.at[pl.ds(base, _CHUNK)], v0, sem),
        pltpu.async_copy(in_hbm.at[pl.ds(base + jnp.int32(_S), _CHUNK)], v1, sem),
        pltpu.async_copy(in_hbm.at[pl.ds(base + jnp.int32(2 * _S), _CHUNK)], v2, sem),
    ]
    for c in copies:
        c.wait()

    def body(g, carry):
        for u in range(_UNROLL):
            off = g * jnp.int32(_UNROLL * _L) + jnp.int32(u * _L)
            idx = pl.ds(pl.multiple_of(off, _L), _L)
            vout[idx] = _hash16(v0[idx], v1[idx], v2[idx])
        return carry

    lax.fori_loop(jnp.int32(0), jnp.int32(_NG // _UNROLL), body, 0)
    pltpu.sync_copy(vout, out_hbm.at[sl])


def _tc_body(a_ref, b_ref, c_ref, o_ref):
    o_ref[...] = _hash16(a_ref[...], b_ref[...], c_ref[...])


@jax.jit
def _crossing(packed, a2, b2, c2):
    run = functools.partial(
        pl.kernel,
        mesh=plsc.VectorSubcoreMesh(core_axis_name="c", subcore_axis_name="s"),
        out_type=jax.ShapeDtypeStruct((_S,), jnp.uint32),
        scratch_types=[pltpu.VMEM((_CHUNK,), jnp.uint32)] * 4
        + [pltpu.SemaphoreType.DMA],
    )(_sc_body)
    sc_out = run(packed)
    tc_out = pl.pallas_call(
        _tc_body,
        out_shape=jax.ShapeDtypeStruct((_T // 128, 128), jnp.uint32),
    )(a2, b2, c2)
    return jnp.concatenate([sc_out, tc_out.reshape(_T)])


def kernel(inp_0, inp_1, inp_2):
    parts = []
    for inp in (inp_0, inp_1, inp_2):
        lo = lax.convert_element_type(inp.reshape(_N), jnp.int32)
        parts.append(lax.bitcast_convert_type(lo, jnp.uint32))
    packed = jnp.concatenate([p[:_S] for p in parts])
    tc_ins = [p[_S:].reshape(_T // 128, 128) for p in parts]
    out = _crossing(packed, *tc_ins)
    return out.astype(jnp.int64).reshape(_N, 1)
